# baseline (speedup 1.0000x reference)
import functools

import jax
import jax.numpy as jnp
from jax import lax
from jax.experimental import pallas as pl
from jax.experimental.pallas import tpu as pltpu
from jax.experimental.pallas import tpu_sc as plsc

F = 8
HP = 2056
NB = 4
W = 512
H = 512
NW = 32
ROWS_PW = H // NW


def _sc_body(uv_ref, tab_ref, out_ref, u_v, v_v, wx_v, wy_v, idx_v,
             rt_v, rt2_v, rts_v, rts2_v, out_v, sem):
    wid = lax.axis_index("s") * 2 + lax.axis_index("c")
    iota16 = lax.iota(jnp.int32, 16)
    row0 = wid * ROWS_PW

    def do_row(k):
        h = row0 + k

        for qb in range(NB):
            pltpu.sync_copy(uv_ref.at[qb, 0, h], u_v.at[qb])
            pltpu.sync_copy(uv_ref.at[qb, 1, h], v_v.at[qb])
            for s in range(4):
                def phase_a(j, carry, qb=qb, s=s):
                    off = (s * 8 + j) * 16
                    yf = u_v[qb, pl.ds(off, 16)] * 2048.0 + 4.0
                    yf = jnp.minimum(jnp.maximum(yf, 0.0), float(HP - 1))
                    yi = jnp.minimum(yf.astype(jnp.int32), HP - 2)
                    xf = v_v[qb, pl.ds(off, 16)] * 2048.0 + 4.0
                    xf = jnp.minimum(jnp.maximum(xf, 0.0), float(HP - 1))
                    xi = jnp.minimum(xf.astype(jnp.int32), HP - 2)
                    wy_v[qb, pl.ds(off, 16)] = yf - yi.astype(jnp.float32)
                    wx_v[qb, pl.ds(off, 16)] = xf - xi.astype(jnp.float32)
                    r0 = yi * HP + xi
                    idx_v[qb, 0, s, pl.ds(j * 16, 16)] = r0
                    idx_v[qb, 1, s, pl.ds(j * 16, 16)] = r0 + 1
                    return carry
                lax.fori_loop(0, 8, phase_a, None)

        for qb in range(NB):
            descs = []
            for s in range(4):
                descs.append(pltpu.async_copy(
                    tab_ref.at[idx_v.at[qb, 0, s]],
                    rt_v.at[pl.ds(s * 128, 128)], sem))
                descs.append(pltpu.async_copy(
                    tab_ref.at[idx_v.at[qb, 1, s]],
                    rt2_v.at[pl.ds(s * 128, 128)], sem))
            for d in descs:
                d.wait()

            def stage(q, carry):
                rts_v[pl.ds(q * 16, 16)] = rt_v[q]
                rts2_v[pl.ds(q * 16, 16)] = rt2_v[q]
                return carry
            lax.fori_loop(0, W, stage, None)

            def combine(g, carry):
                off = g * 16
                qvec = (off + iota16) * 16
                for wb in range(NB):
                    wx = wx_v[wb, pl.ds(off, 16)]
                    wy = wy_v[wb, pl.ds(off, 16)]
                    for ci in range(2):
                        ch = 2 * wb + ci
                        a = plsc.load_gather(rts_v, [qvec + ch])
                        cc = plsc.load_gather(rts_v, [qvec + (ch + 8)])
                        b_ = plsc.load_gather(rts2_v, [qvec + ch])
                        dd = plsc.load_gather(rts2_v, [qvec + (ch + 8)])
                        top = a + wx * (b_ - a)
                        bot = cc + wx * (dd - cc)
                        out_v[ch, pl.ds(off, 16)] = top + wy * (bot - top)
                return carry
            lax.fori_loop(0, W // 16, combine, None)

            for ch in range(F):
                oi = ch * NB + qb
                pltpu.sync_copy(out_v.at[ch],
                                out_ref.at[oi // F, oi % F, h])

    def row_loop(k, carry):
        do_row(k)
        return carry
    lax.fori_loop(0, ROWS_PW, row_loop, None)


@jax.jit
def kernel(uv, feature_img):
    t = jnp.transpose(feature_img, (1, 2, 0))
    tab = jnp.concatenate([t[:-1], t[1:]], axis=2).reshape((HP - 1) * HP, 2 * F)
    run = functools.partial(
        pl.kernel,
        out_type=jax.ShapeDtypeStruct((NB, F, H, W), jnp.float32),
        mesh=plsc.VectorSubcoreMesh(core_axis_name="c", subcore_axis_name="s"),
        compiler_params=pltpu.CompilerParams(
            needs_layout_passes=False, use_tc_tiling_on_sc=False),
        scratch_types=[
            pltpu.VMEM((NB, W), jnp.float32),
            pltpu.VMEM((NB, W), jnp.float32),
            pltpu.VMEM((NB, W), jnp.float32),
            pltpu.VMEM((NB, W), jnp.float32),
            pltpu.VMEM((NB, 2, 4, 128), jnp.int32),
            pltpu.VMEM((W, 2 * F), jnp.float32),
            pltpu.VMEM((W, 2 * F), jnp.float32),
            pltpu.VMEM((W * 2 * F,), jnp.float32),
            pltpu.VMEM((W * 2 * F,), jnp.float32),
            pltpu.VMEM((F, W), jnp.float32),
            pltpu.SemaphoreType.DMA,
        ],
    )(_sc_body)
    return run(uv, tab)
